# Initial kernel scaffold; baseline (speedup 1.0000x reference)
import jax
import jax.numpy as jnp
from jax.experimental import pallas as pl


def kernel(input_ids, params):
    raise NotImplementedError("write your pallas kernel here")


# speedup vs baseline: 1.4230x; 1.4230x over previous
import functools

import jax
import jax.numpy as jnp
from jax.experimental import pallas as pl
from jax.experimental.pallas import tpu as pltpu

B, S, D = 1, 2048, 1024
H, KV, HD = 16, 8, 64
E, TOPK, F = 8, 2, 1024
L = 2
EPS = 1e-6
THETA = 10000.0

BQ = 256
BS_MOE = 512


def _rms(x, w):
    return x * jax.lax.rsqrt(jnp.mean(x * x, axis=-1, keepdims=True) + EPS) * w


def _qkv_body(h_ref, ln_ref, wq_ref, wk_ref, wv_ref, cos_ref, sin_ref,
              q_ref, k_ref, v_ref):
    h = h_ref[...]
    r = _rms(h, ln_ref[...]).astype(jnp.bfloat16)
    cos = cos_ref[...]
    sin = sin_ref[...]

    def rope(x, nheads):
        cf = jnp.concatenate([cos] * nheads, axis=1)
        sf = jnp.concatenate([sin] * nheads, axis=1)
        lane = jax.lax.broadcasted_iota(jnp.int32, x.shape, 1) % HD
        first = lane < (HD // 2)
        xm = pltpu.roll(x, x.shape[1] - HD // 2, 1)
        xp = pltpu.roll(x, HD // 2, 1)
        rot = jnp.where(first, -xm, xp)
        return x * cf + rot * sf

    q = jnp.dot(r, wq_ref[...], preferred_element_type=jnp.float32)
    k = jnp.dot(r, wk_ref[...], preferred_element_type=jnp.float32)
    v = jnp.dot(r, wv_ref[...], preferred_element_type=jnp.float32)
    q_ref[...] = rope(q, H).astype(jnp.bfloat16)
    k_ref[...] = rope(k, KV).astype(jnp.bfloat16)
    v_ref[...] = v.astype(jnp.bfloat16)


def _qkv(h, ln1, wq, wk, wv, cos, sin):
    grid = (S // BQ,)
    return pl.pallas_call(
        _qkv_body,
        grid=grid,
        in_specs=[
            pl.BlockSpec((BQ, D), lambda i: (i, 0)),
            pl.BlockSpec((1, D), lambda i: (0, 0)),
            pl.BlockSpec((D, H * HD), lambda i: (0, 0)),
            pl.BlockSpec((D, KV * HD), lambda i: (0, 0)),
            pl.BlockSpec((D, KV * HD), lambda i: (0, 0)),
            pl.BlockSpec((BQ, HD), lambda i: (i, 0)),
            pl.BlockSpec((BQ, HD), lambda i: (i, 0)),
        ],
        out_specs=[
            pl.BlockSpec((BQ, H * HD), lambda i: (i, 0)),
            pl.BlockSpec((BQ, KV * HD), lambda i: (i, 0)),
            pl.BlockSpec((BQ, KV * HD), lambda i: (i, 0)),
        ],
        out_shape=[
            jax.ShapeDtypeStruct((S, H * HD), jnp.bfloat16),
            jax.ShapeDtypeStruct((S, KV * HD), jnp.bfloat16),
            jax.ShapeDtypeStruct((S, KV * HD), jnp.bfloat16),
        ],
        compiler_params=pltpu.CompilerParams(
            dimension_semantics=("arbitrary",)),
    )(h, ln1, wq, wk, wv, cos, sin)


def _attn_body(q_ref, k_ref, v_ref, o_ref):
    i = pl.program_id(1)
    q = q_ref[0]
    k = k_ref[0]
    s = jax.lax.dot_general(q, k, (((1,), (1,)), ((), ())),
                            preferred_element_type=jnp.float32)
    row = i * BQ + jax.lax.broadcasted_iota(jnp.int32, s.shape, 0)
    col = jax.lax.broadcasted_iota(jnp.int32, s.shape, 1)
    s = s * (1.0 / (HD ** 0.5)) + jnp.where(col <= row, 0.0, -1e9)
    m = jnp.max(s, axis=-1, keepdims=True)
    p = jnp.exp(s - m)
    p = p / jnp.sum(p, axis=-1, keepdims=True)
    o = jnp.dot(p.astype(jnp.bfloat16), v_ref[0],
                preferred_element_type=jnp.float32)
    o_ref[0] = o.astype(jnp.bfloat16)


def _attn(q, k, v):
    grid = (H, S // BQ)
    g = H // KV
    return pl.pallas_call(
        _attn_body,
        grid=grid,
        in_specs=[
            pl.BlockSpec((1, BQ, HD), lambda h, i: (h, i, 0)),
            pl.BlockSpec((1, S, HD), lambda h, i: (h // g, 0, 0)),
            pl.BlockSpec((1, S, HD), lambda h, i: (h // g, 0, 0)),
        ],
        out_specs=pl.BlockSpec((1, BQ, HD), lambda h, i: (h, i, 0)),
        out_shape=jax.ShapeDtypeStruct((H, S, HD), jnp.bfloat16),
        compiler_params=pltpu.CompilerParams(
            dimension_semantics=("arbitrary", "arbitrary")),
    )(q, k, v)


def _post_body(a_ref, wo_ref, h_ref, ln_ref, wg_ref, h2_ref, r2_ref, wf_ref):
    h2 = h_ref[...] + jnp.dot(a_ref[...], wo_ref[...],
                              preferred_element_type=jnp.float32)
    h2_ref[...] = h2
    r2 = _rms(h2, ln_ref[...])
    r2_ref[...] = r2.astype(jnp.bfloat16)
    logits = jnp.dot(r2, wg_ref[...], preferred_element_type=jnp.float32)
    mx = jnp.max(logits, axis=-1, keepdims=True)
    ex = jnp.exp(logits - mx)
    probs = ex / jnp.sum(ex, axis=-1, keepdims=True)
    eidx = jax.lax.broadcasted_iota(jnp.int32, probs.shape, 1)
    m1 = jnp.max(probs, axis=-1, keepdims=True)
    i1 = jnp.min(jnp.where(probs == m1, eidx, E), axis=-1, keepdims=True)
    mask1 = eidx == i1
    pm = jnp.where(mask1, -1.0, probs)
    m2 = jnp.max(pm, axis=-1, keepdims=True)
    i2 = jnp.min(jnp.where(pm == m2, eidx, E), axis=-1, keepdims=True)
    mask2 = eidx == i2
    denom = m1 + m2
    wf_ref[...] = (jnp.where(mask1, m1, 0.0) + jnp.where(mask2, m2, 0.0)) / denom


def _post(a, wo, h, ln2, wg):
    grid = (S // BQ,)
    return pl.pallas_call(
        _post_body,
        grid=grid,
        in_specs=[
            pl.BlockSpec((BQ, H * HD), lambda i: (i, 0)),
            pl.BlockSpec((H * HD, D), lambda i: (0, 0)),
            pl.BlockSpec((BQ, D), lambda i: (i, 0)),
            pl.BlockSpec((1, D), lambda i: (0, 0)),
            pl.BlockSpec((D, E), lambda i: (0, 0)),
        ],
        out_specs=[
            pl.BlockSpec((BQ, D), lambda i: (i, 0)),
            pl.BlockSpec((BQ, D), lambda i: (i, 0)),
            pl.BlockSpec((BQ, E), lambda i: (i, 0)),
        ],
        out_shape=[
            jax.ShapeDtypeStruct((S, D), jnp.float32),
            jax.ShapeDtypeStruct((S, D), jnp.bfloat16),
            jax.ShapeDtypeStruct((S, E), jnp.float32),
        ],
        compiler_params=pltpu.CompilerParams(
            dimension_semantics=("arbitrary",)),
    )(a, wo, h, ln2, wg)


def _moe_body(x_ref, w1_ref, w3_ref, w2_ref, wf_ref, h2_ref, out_ref):
    e = pl.program_id(1)
    x = x_ref[...]
    t1 = jnp.dot(x, w1_ref[0], preferred_element_type=jnp.float32)
    t3 = jnp.dot(x, w3_ref[0], preferred_element_type=jnp.float32)
    t = (t1 * jax.lax.logistic(t1) * t3).astype(jnp.bfloat16)
    ex = jnp.dot(t, w2_ref[0], preferred_element_type=jnp.float32)
    eidx = jax.lax.broadcasted_iota(jnp.int32, wf_ref.shape, 1)
    we = jnp.sum(jnp.where(eidx == e, wf_ref[...], 0.0), axis=-1,
                 keepdims=True)

    @pl.when(e == 0)
    def _():
        out_ref[...] = h2_ref[...] + we * ex

    @pl.when(e > 0)
    def _():
        out_ref[...] = out_ref[...] + we * ex


def _moe(x, w1, w3, w2, wf, h2):
    grid = (S // BS_MOE, E)
    return pl.pallas_call(
        _moe_body,
        grid=grid,
        in_specs=[
            pl.BlockSpec((BS_MOE, D), lambda i, e: (i, 0)),
            pl.BlockSpec((1, D, F), lambda i, e: (e, 0, 0)),
            pl.BlockSpec((1, D, F), lambda i, e: (e, 0, 0)),
            pl.BlockSpec((1, F, D), lambda i, e: (e, 0, 0)),
            pl.BlockSpec((BS_MOE, E), lambda i, e: (i, 0)),
            pl.BlockSpec((BS_MOE, D), lambda i, e: (i, 0)),
        ],
        out_specs=pl.BlockSpec((BS_MOE, D), lambda i, e: (i, 0)),
        out_shape=jax.ShapeDtypeStruct((S, D), jnp.float32),
        compiler_params=pltpu.CompilerParams(
            dimension_semantics=("parallel", "arbitrary")),
    )(x, w1, w3, w2, wf, h2)


def _fln_body(h_ref, ln_ref, o_ref):
    o_ref[...] = _rms(h_ref[...], ln_ref[...])


def _fln(h, ln):
    return pl.pallas_call(
        _fln_body,
        grid=(S // BQ,),
        in_specs=[
            pl.BlockSpec((BQ, D), lambda i: (i, 0)),
            pl.BlockSpec((1, D), lambda i: (0, 0)),
        ],
        out_specs=pl.BlockSpec((BQ, D), lambda i: (i, 0)),
        out_shape=jax.ShapeDtypeStruct((S, D), jnp.float32),
        compiler_params=pltpu.CompilerParams(
            dimension_semantics=("arbitrary",)),
    )(h, ln)


def kernel(input_ids, params):
    x = input_ids.reshape(S, D)

    pos = jnp.arange(S, dtype=jnp.float32)
    inv_freq = 1.0 / (THETA ** (jnp.arange(0, HD, 2, dtype=jnp.float32) / HD))
    freqs = jnp.outer(pos, inv_freq)
    emb = jnp.concatenate([freqs, freqs], axis=-1)
    cos = jnp.cos(emb)
    sin = jnp.sin(emb)

    h = x
    for l in range(L):
        p = params['layer_%d' % l]
        wq = p['wq'].astype(jnp.bfloat16)
        wk = p['wk'].astype(jnp.bfloat16)
        wv = p['wv'].astype(jnp.bfloat16)
        wo = p['wo'].astype(jnp.bfloat16)
        w1 = p['w1'].astype(jnp.bfloat16)
        w3 = p['w3'].astype(jnp.bfloat16)
        w2 = p['w2'].astype(jnp.bfloat16)

        q2, k2, v2 = _qkv(h, p['ln1'].reshape(1, D), wq, wk, wv, cos, sin)
        q = q2.reshape(S, H, HD).transpose(1, 0, 2)
        k = k2.reshape(S, KV, HD).transpose(1, 0, 2)
        v = v2.reshape(S, KV, HD).transpose(1, 0, 2)
        o = _attn(q, k, v)
        a = o.transpose(1, 0, 2).reshape(S, H * HD)
        h2, r2, wf = _post(a, wo, h, p['ln2'].reshape(1, D), p['wg'])
        h = _moe(r2, w1, w3, w2, wf, h2)

    out = _fln(h, params['final_ln'].reshape(1, D))
    return out.reshape(B, S, D)
